# baseline (speedup 1.0000x reference)
import functools

import jax
import jax.numpy as jnp
from jax import lax
from jax.experimental import pallas as pl
from jax.experimental.pallas import tpu as pltpu
from jax.experimental.pallas import tpu_sc as plsc

NUM_SAMPLES = 4096
SEQ = 50
NUM_ROWS = NUM_SAMPLES * SEQ
DIM = 128
NC, NS = 2, 16
NW = NC * NS
NPART = 4
P_SAMPLES = NUM_SAMPLES // NPART
P_ROWS = P_SAMPLES * SEQ
B_PER_W = P_ROWS // NW
CHUNK = 400
N_CHUNKS = B_PER_W // CHUNK
NBUF = 2

_mesh = plsc.VectorSubcoreMesh(
    core_axis_name="c", subcore_axis_name="s", num_cores=NC, num_subcores=NS
)


@functools.partial(
    pl.kernel,
    out_type=jax.ShapeDtypeStruct((P_ROWS, DIM), jnp.float32),
    mesh=_mesh,
    scratch_types=[
        pltpu.VMEM((B_PER_W,), jnp.int32),
        pltpu.VMEM((NBUF, CHUNK, DIM), jnp.float32),
        [pltpu.SemaphoreType.DMA] * NBUF,
        [pltpu.SemaphoreType.DMA] * NBUF,
    ],
)
def _emb_lookup(idx_hbm, table_hbm, out_hbm, idx_v, rows_v, gsem, wsem):
    wid = lax.axis_index("s") * NC + lax.axis_index("c")
    base = wid * B_PER_W
    pltpu.sync_copy(idx_hbm.at[pl.ds(base, B_PER_W)], idx_v)

    def ids_of(c):
        return idx_v.at[pl.ds(c * CHUNK, CHUNK)]

    def out_of(c):
        return out_hbm.at[pl.ds(base + c * CHUNK, CHUNK)]

    for b in range(NBUF):
        pltpu.async_copy(table_hbm.at[ids_of(b)], rows_v.at[b], gsem[b])

    for c in range(N_CHUNKS):
        b = c % NBUF
        pltpu.make_async_copy(table_hbm.at[ids_of(c)], rows_v.at[b], gsem[b]).wait()
        pltpu.async_copy(rows_v.at[b], out_of(c), wsem[b])
        nxt = c + NBUF
        if nxt < N_CHUNKS:
            pltpu.make_async_copy(rows_v.at[b], out_of(c), wsem[b]).wait()
            pltpu.async_copy(table_hbm.at[ids_of(nxt)], rows_v.at[b], gsem[b])

    for c in range(max(N_CHUNKS - NBUF, 0), N_CHUNKS):
        b = c % NBUF
        pltpu.make_async_copy(rows_v.at[b], out_of(c), wsem[b]).wait()


def kernel(token_ids, embeddings):
    flat_ids = token_ids.reshape(NUM_ROWS).astype(jnp.int32)
    out = jnp.zeros((NUM_SAMPLES, SEQ, DIM), jnp.float32)
    for p in range(NPART):
        part = _emb_lookup(flat_ids[p * P_ROWS:(p + 1) * P_ROWS], embeddings)
        out = lax.dynamic_update_slice(
            out, part.reshape(P_SAMPLES, SEQ, DIM), (p * P_SAMPLES, 0, 0)
        )
    return out


# speedup vs baseline: 2.3095x; 2.3095x over previous
import functools

import jax
import jax.numpy as jnp
from jax import lax
from jax.experimental import pallas as pl
from jax.experimental.pallas import tpu as pltpu
from jax.experimental.pallas import tpu_sc as plsc

NUM_SAMPLES = 4096
SEQ = 50
NUM_ROWS = NUM_SAMPLES * SEQ
DIM = 128
NC, NS = 2, 16
NW = NC * NS
S_PER_W = NUM_SAMPLES // NW
B_PER_W = S_PER_W * SEQ
S_CHUNK = 8
CHUNK = S_CHUNK * SEQ
N_CHUNKS = S_PER_W // S_CHUNK
NBUF = 2

_mesh = plsc.VectorSubcoreMesh(
    core_axis_name="c", subcore_axis_name="s", num_cores=NC, num_subcores=NS
)


@functools.partial(
    pl.kernel,
    out_type=jax.ShapeDtypeStruct((NUM_SAMPLES, SEQ, DIM), jnp.float32),
    mesh=_mesh,
    compiler_params=pltpu.CompilerParams(
        use_tc_tiling_on_sc=True, needs_layout_passes=True
    ),
    scratch_types=[
        pltpu.VMEM((B_PER_W,), jnp.int32),
        pltpu.VMEM((NBUF, CHUNK, DIM), jnp.float32),
        [pltpu.SemaphoreType.DMA] * NBUF,
        [pltpu.SemaphoreType.DMA] * NBUF,
    ],
)
def _emb_lookup(idx_hbm, table_hbm, out_hbm, idx_v, rows_v, gsem, wsem):
    wid = lax.axis_index("s") * NC + lax.axis_index("c")
    base = wid * B_PER_W
    s_base = wid * S_PER_W
    pltpu.sync_copy(idx_hbm.at[pl.ds(base, B_PER_W)], idx_v)

    def ids_of(c):
        return idx_v.at[pl.ds(c * CHUNK, CHUNK)]

    def writes_of(c, b):
        i0 = s_base + c * S_CHUNK
        return [
            (rows_v.at[b, pl.ds(s * SEQ, SEQ)], out_hbm.at[i0 + s])
            for s in range(S_CHUNK)
        ]

    for b in range(NBUF):
        pltpu.async_copy(table_hbm.at[ids_of(b)], rows_v.at[b], gsem[b])

    for c in range(N_CHUNKS):
        b = c % NBUF
        pltpu.make_async_copy(table_hbm.at[ids_of(c)], rows_v.at[b], gsem[b]).wait()
        for src, dst in writes_of(c, b):
            pltpu.async_copy(src, dst, wsem[b])
        nxt = c + NBUF
        if nxt < N_CHUNKS:
            for src, dst in writes_of(c, b):
                pltpu.make_async_copy(src, dst, wsem[b]).wait()
            pltpu.async_copy(table_hbm.at[ids_of(nxt)], rows_v.at[b], gsem[b])

    for c in range(N_CHUNKS - NBUF, N_CHUNKS):
        b = c % NBUF
        for src, dst in writes_of(c, b):
            pltpu.make_async_copy(src, dst, wsem[b]).wait()


def kernel(token_ids, embeddings):
    flat_ids = token_ids.reshape(NUM_ROWS).astype(jnp.int32)
    return _emb_lookup(flat_ids, embeddings)
